# baseline (device time: 14946 ns/iter reference)
import jax
import jax.numpy as jnp
from jax import lax
from jax.experimental import pallas as pl
from jax.experimental.pallas import tpu as pltpu

N_DEV = 4
B, SQ, SKV, HQ, DH = 2, 128, 512, 4, 64
SKV_LOC = SKV // N_DEV
HD = HQ * DH
SCALE = 0.125

NN = (((1,), (0,)), ((), ()))
NT = (((1,), (1,)), ((), ()))


def kernel(x, Wq, K_ext, V_ext, Wo):
    K2 = K_ext.reshape(B, SKV_LOC, HD)
    V2 = V_ext.reshape(B, SKV_LOC, HD)

    def body(x_ref, wq_ref, k_ref, v_ref, wo_ref, out_ref,
             kv01_ref, ctx_ref, send_sems, recv_sems):
        my = lax.axis_index("i")

        barrier_sem = pltpu.get_barrier_semaphore()

        @pl.when(my == 0)
        def _():
            pl.semaphore_signal(
                barrier_sem, inc=1,
                device_id=(1,), device_id_type=pl.DeviceIdType.MESH)

        @pl.when(my == 1)
        def _():
            pl.semaphore_signal(
                barrier_sem, inc=1,
                device_id=(0,), device_id_type=pl.DeviceIdType.MESH)

        @pl.when(my >= 2)
        def _():
            for tgt in (0, 1):
                pl.semaphore_signal(
                    barrier_sem, inc=1,
                    device_id=(tgt,), device_id_type=pl.DeviceIdType.MESH)

        @pl.when(my < 2)
        def _():
            pl.semaphore_wait(barrier_sem, 3)

        def do_broadcast(slot, targets):
            kv01_ref[slot, 0] = k_ref[...].astype(jnp.bfloat16)
            kv01_ref[slot, 1] = v_ref[...].astype(jnp.bfloat16)
            for j, tgt in enumerate(targets):
                pltpu.make_async_remote_copy(
                    src_ref=kv01_ref.at[slot], dst_ref=kv01_ref.at[slot],
                    send_sem=send_sems.at[j], recv_sem=recv_sems.at[slot],
                    device_id=(tgt,), device_id_type=pl.DeviceIdType.MESH,
                ).start()

        @pl.when(my == 0)
        def _():
            do_broadcast(0, (1, 2, 3))

        @pl.when(my == 1)
        def _():
            do_broadcast(1, (0, 2, 3))

        wq = wq_ref[...].astype(jnp.bfloat16)
        wo = wo_ref[...].astype(jnp.bfloat16)
        qi = lax.broadcasted_iota(jnp.int32, (SQ, SKV_LOC), 0)
        ki = lax.broadcasted_iota(jnp.int32, (SQ, SKV_LOC), 1)
        tri = ki <= qi
        qbs = []
        for b in range(B):
            xb = x_ref[b].astype(jnp.bfloat16)
            qb = lax.dot_general(
                xb, wq, NN, preferred_element_type=jnp.float32) * SCALE
            qbs.append(qb.astype(jnp.bfloat16))

        def wait_recv_slot(s):
            pltpu.make_async_remote_copy(
                src_ref=kv01_ref.at[s], dst_ref=kv01_ref.at[s],
                send_sem=send_sems.at[0], recv_sem=recv_sems.at[s],
                device_id=(0,), device_id_type=pl.DeviceIdType.MESH,
            ).wait_recv()

        def block_partial(slot, b, h):
            kb = kv01_ref[slot, 0, b]
            vb = kv01_ref[slot, 1, b]
            kbh = lax.slice(kb, (0, h * DH), (SKV_LOC, (h + 1) * DH))
            vbh = lax.slice(vb, (0, h * DH), (SKV_LOC, (h + 1) * DH))
            qbh = lax.slice(qbs[b], (0, h * DH), (SQ, (h + 1) * DH))
            s = lax.dot_general(
                qbh, kbh, NT, preferred_element_type=jnp.float32)
            w = jnp.exp(s)
            if slot == 1:
                w = jnp.where(tri, w, 0.0)
            denom = jnp.sum(w, axis=1, keepdims=True)
            ctx = lax.dot_general(
                w.astype(jnp.bfloat16), vbh, NN,
                preferred_element_type=jnp.float32)
            return ctx, denom

        def attn(first, second, wait_first):
            if wait_first:
                wait_recv_slot(first)
            acc = {}
            for b in range(B):
                for h in range(HQ):
                    acc[(b, h)] = block_partial(first, b, h)
            wait_recv_slot(second)
            for b in range(B):
                for h in range(HQ):
                    c0, d0 = acc[(b, h)]
                    c1, d1 = block_partial(second, b, h)
                    ctx = (c0 + c1) / (d0 + d1)
                    ctx_ref[b, :, pl.ds(h * DH, DH)] = ctx.astype(jnp.bfloat16)

        @pl.when(my == 0)
        def _():
            attn(0, 1, wait_first=False)

        @pl.when(my == 1)
        def _():
            attn(1, 0, wait_first=False)

        @pl.when(my == 2)
        def _():
            attn(1, 0, wait_first=True)

        @pl.when(my == 3)
        def _():
            attn(0, 1, wait_first=True)

        cb = jnp.concatenate([ctx_ref[0], ctx_ref[1]], axis=0)
        o = lax.dot_general(cb, wo, NN, preferred_element_type=jnp.float32)
        out_ref[0] = lax.slice(o, (0, 0), (SQ, SKV))
        out_ref[1] = lax.slice(o, (SQ, 0), (2 * SQ, SKV))

        def wait_sends(slot):
            for j in range(3):
                pltpu.make_async_remote_copy(
                    src_ref=kv01_ref.at[slot], dst_ref=kv01_ref.at[slot],
                    send_sem=send_sems.at[j], recv_sem=recv_sems.at[slot],
                    device_id=(0,), device_id_type=pl.DeviceIdType.MESH,
                ).wait_send()

        @pl.when(my == 0)
        def _():
            wait_sends(0)

        @pl.when(my == 1)
        def _():
            wait_sends(1)

    return pl.pallas_call(
        body,
        out_shape=jax.ShapeDtypeStruct((B, SQ, SKV), jnp.float32),
        in_specs=[pl.BlockSpec(memory_space=pltpu.VMEM)] * 5,
        out_specs=pl.BlockSpec(memory_space=pltpu.VMEM),
        scratch_shapes=[
            pltpu.VMEM((2, 2, B, SKV_LOC, HD), jnp.bfloat16),
            pltpu.VMEM((B, SQ, HD), jnp.bfloat16),
            pltpu.SemaphoreType.DMA((3,)),
            pltpu.SemaphoreType.DMA((2,)),
        ],
        compiler_params=pltpu.CompilerParams(collective_id=0),
    )(x, Wq, K2, V2, Wo)


# device time: 14249 ns/iter; 1.0489x vs baseline; 1.0489x over previous
import jax
import jax.numpy as jnp
from jax import lax
from jax.experimental import pallas as pl
from jax.experimental.pallas import tpu as pltpu

N_DEV = 4
B, SQ, SKV, HQ, DH = 2, 128, 512, 4, 64
SKV_LOC = SKV // N_DEV
HD = HQ * DH
HH = HD // 2
SCALE = 0.125

NN = (((1,), (0,)), ((), ()))
NT = (((1,), (1,)), ((), ()))


def kernel(x, Wq, K_ext, V_ext, Wo):
    K2 = K_ext.reshape(B, SKV_LOC, HD)
    V2 = V_ext.reshape(B, SKV_LOC, HD)

    def body(x_ref, wq_ref, k_ref, v_ref, wo_ref, out_ref,
             kvstage_ref, kvq_ref, ctxq_ref,
             ssem1, ssem2, rsem1, rsem2):
        my = lax.axis_index("i")

        barrier_sem = pltpu.get_barrier_semaphore()

        @pl.when(my == 0)
        def _():
            pl.semaphore_signal(
                barrier_sem, inc=1,
                device_id=(1,), device_id_type=pl.DeviceIdType.MESH)

        @pl.when(my == 1)
        def _():
            pl.semaphore_signal(
                barrier_sem, inc=1,
                device_id=(0,), device_id_type=pl.DeviceIdType.MESH)

        @pl.when(my >= 2)
        def _():
            for tgt in (0, 1):
                pl.semaphore_signal(
                    barrier_sem, inc=1,
                    device_id=(tgt,), device_id_type=pl.DeviceIdType.MESH)

        @pl.when(my < 2)
        def _():
            pl.semaphore_wait(barrier_sem, 3)

        wq = wq_ref[...].astype(jnp.bfloat16)
        wo = wo_ref[...].astype(jnp.bfloat16)
        qi = lax.broadcasted_iota(jnp.int32, (SQ, SKV_LOC), 0)
        ki = lax.broadcasted_iota(jnp.int32, (SQ, SKV_LOC), 1)
        tri = ki <= qi

        def recv_block(slot):
            for kv in (0, 1):
                pltpu.make_async_remote_copy(
                    src_ref=kvq_ref.at[slot, kv], dst_ref=kvq_ref.at[slot, kv],
                    send_sem=ssem1.at[0], recv_sem=rsem1.at[slot * 2 + kv],
                    device_id=(0,), device_id_type=pl.DeviceIdType.MESH,
                ).wait_recv()

        def run_role(d):
            bq, half = d // 2, d % 2
            off = half * HH

            if d in (0, 1):
                kvstage_ref[0] = k_ref[...].astype(jnp.bfloat16)
                kvstage_ref[1] = v_ref[...].astype(jnp.bfloat16)
                for kv in (0, 1):
                    kvq_ref[d, kv] = lax.slice(
                        kvstage_ref[kv, bq], (0, off), (SKV_LOC, off + HH))
                jj = 0
                for t in range(N_DEV):
                    if t == d:
                        continue
                    bt, ht = t // 2, t % 2
                    offt = ht * HH
                    for kv in (0, 1):
                        pltpu.make_async_remote_copy(
                            src_ref=kvstage_ref.at[
                                kv, bt, :, pl.ds(offt, HH)],
                            dst_ref=kvq_ref.at[d, kv],
                            send_sem=ssem1.at[jj],
                            recv_sem=rsem1.at[d * 2 + kv],
                            device_id=(t,),
                            device_id_type=pl.DeviceIdType.MESH,
                        ).start()
                        jj += 1

            wqh = lax.slice(wq, (0, off), (512, off + HH))
            xb = x_ref[bq].astype(jnp.bfloat16)
            q = (lax.dot_general(
                xb, wqh, NN, preferred_element_type=jnp.float32)
                * SCALE).astype(jnp.bfloat16)

            first, second = {0: (0, 1), 1: (1, 0), 2: (1, 0), 3: (0, 1)}[d]
            if d >= 2:
                recv_block(first)

            def blk(slot, h_local):
                kk = kvq_ref[slot, 0]
                vv = kvq_ref[slot, 1]
                sl = (0, h_local * DH)
                sh = (SKV_LOC, (h_local + 1) * DH)
                kbh = lax.slice(kk, sl, sh)
                vbh = lax.slice(vv, sl, sh)
                qh = lax.slice(q, (0, h_local * DH), (SQ, (h_local + 1) * DH))
                s = lax.dot_general(
                    qh, kbh, NT, preferred_element_type=jnp.float32)
                w = jnp.exp(s)
                if slot == 1:
                    w = jnp.where(tri, w, 0.0)
                denom = jnp.sum(w, axis=1, keepdims=True)
                ctx = lax.dot_general(
                    w.astype(jnp.bfloat16), vbh, NN,
                    preferred_element_type=jnp.float32)
                return ctx, denom

            acc = [blk(first, hl) for hl in range(2)]
            recv_block(second)
            for hl in range(2):
                c0, d0_ = acc[hl]
                c1, d1_ = blk(second, hl)
                ctx = (c0 + c1) / (d0_ + d1_)
                ctxq_ref[d, :, pl.ds(hl * DH, DH)] = ctx.astype(jnp.bfloat16)

            for j, t in enumerate(tt for tt in range(N_DEV) if tt != d):
                pltpu.make_async_remote_copy(
                    src_ref=ctxq_ref.at[d], dst_ref=ctxq_ref.at[d],
                    send_sem=ssem2.at[j], recv_sem=rsem2.at[d],
                    device_id=(t,), device_id_type=pl.DeviceIdType.MESH,
                ).start()
            for s in range(N_DEV):
                if s == d:
                    continue
                pltpu.make_async_remote_copy(
                    src_ref=ctxq_ref.at[s], dst_ref=ctxq_ref.at[s],
                    send_sem=ssem2.at[0], recv_sem=rsem2.at[s],
                    device_id=(0,), device_id_type=pl.DeviceIdType.MESH,
                ).wait_recv()

        for d in range(N_DEV):
            @pl.when(my == d)
            def _(d=d):
                run_role(d)

        cb = jnp.concatenate(
            [jnp.concatenate([ctxq_ref[0], ctxq_ref[1]], axis=1),
             jnp.concatenate([ctxq_ref[2], ctxq_ref[3]], axis=1)],
            axis=0)
        o = lax.dot_general(cb, wo, NN, preferred_element_type=jnp.float32)
        out_ref[0] = lax.slice(o, (0, 0), (SQ, SKV))
        out_ref[1] = lax.slice(o, (SQ, 0), (2 * SQ, SKV))

        def wait_sends(d):
            if d in (0, 1):
                for j in range(6):
                    pltpu.make_async_remote_copy(
                        src_ref=kvq_ref.at[0, 0], dst_ref=kvq_ref.at[0, 0],
                        send_sem=ssem1.at[j], recv_sem=rsem1.at[0],
                        device_id=(0,), device_id_type=pl.DeviceIdType.MESH,
                    ).wait_send()
            for j in range(3):
                pltpu.make_async_remote_copy(
                    src_ref=ctxq_ref.at[0], dst_ref=ctxq_ref.at[0],
                    send_sem=ssem2.at[j], recv_sem=rsem2.at[0],
                    device_id=(0,), device_id_type=pl.DeviceIdType.MESH,
                ).wait_send()

        for d in range(N_DEV):
            @pl.when(my == d)
            def _(d=d):
                wait_sends(d)

    return pl.pallas_call(
        body,
        out_shape=jax.ShapeDtypeStruct((B, SQ, SKV), jnp.float32),
        in_specs=[pl.BlockSpec(memory_space=pltpu.VMEM)] * 5,
        out_specs=pl.BlockSpec(memory_space=pltpu.VMEM),
        scratch_shapes=[
            pltpu.VMEM((2, B, SKV_LOC, HD), jnp.bfloat16),
            pltpu.VMEM((2, 2, SKV_LOC, HH), jnp.bfloat16),
            pltpu.VMEM((N_DEV, SQ, HH), jnp.bfloat16),
            pltpu.SemaphoreType.DMA((6,)),
            pltpu.SemaphoreType.DMA((3,)),
            pltpu.SemaphoreType.DMA((4,)),
            pltpu.SemaphoreType.DMA((4,)),
        ],
        compiler_params=pltpu.CompilerParams(collective_id=0),
    )(x, Wq, K2, V2, Wo)


# device time: 14203 ns/iter; 1.0523x vs baseline; 1.0032x over previous
import jax
import jax.numpy as jnp
from jax import lax
from jax.experimental import pallas as pl
from jax.experimental.pallas import tpu as pltpu

N_DEV = 4
B, SQ, SKV, HQ, DH = 2, 128, 512, 4, 64
SKV_LOC = SKV // N_DEV
HD = HQ * DH
HH = HD // 2
SCALE = 0.125

NN = (((1,), (0,)), ((), ()))
NT = (((1,), (1,)), ((), ()))


def kernel(x, Wq, K_ext, V_ext, Wo):
    K2 = K_ext.reshape(B, SKV_LOC, HD)
    V2 = V_ext.reshape(B, SKV_LOC, HD)

    def body(x_ref, wq_ref, k_ref, v_ref, wo_ref, out_ref,
             sendstage_ref, kvq_ref, ctxq_ref,
             ssem1, ssem2, rsem1, rsem2):
        my = lax.axis_index("i")

        barrier_sem = pltpu.get_barrier_semaphore()

        @pl.when(my == 0)
        def _():
            pl.semaphore_signal(
                barrier_sem, inc=1,
                device_id=(1,), device_id_type=pl.DeviceIdType.MESH)

        @pl.when(my == 1)
        def _():
            pl.semaphore_signal(
                barrier_sem, inc=1,
                device_id=(0,), device_id_type=pl.DeviceIdType.MESH)

        @pl.when(my >= 2)
        def _():
            for tgt in (0, 1):
                pl.semaphore_signal(
                    barrier_sem, inc=1,
                    device_id=(tgt,), device_id_type=pl.DeviceIdType.MESH)

        @pl.when(my < 2)
        def _():
            pl.semaphore_wait(barrier_sem, 3)

        wq = wq_ref[...].astype(jnp.bfloat16)
        wo = wo_ref[...].astype(jnp.bfloat16)
        qi = lax.broadcasted_iota(jnp.int32, (SQ, SKV_LOC), 0)
        ki = lax.broadcasted_iota(jnp.int32, (SQ, SKV_LOC), 1)
        tri = ki <= qi

        def recv_block(slot):
            pltpu.make_async_remote_copy(
                src_ref=kvq_ref.at[slot], dst_ref=kvq_ref.at[slot],
                send_sem=ssem1.at[0], recv_sem=rsem1.at[slot],
                device_id=(0,), device_id_type=pl.DeviceIdType.MESH,
            ).wait_recv()

        def run_role(d):
            bq, half = d // 2, d % 2
            off = half * HH
            others = [t for t in range(N_DEV) if t != d]

            if d in (0, 1):
                kb = [k_ref[b].astype(jnp.bfloat16) for b in range(B)]
                vb = [v_ref[b].astype(jnp.bfloat16) for b in range(B)]
                for i, t in enumerate(others):
                    bt, offt = t // 2, (t % 2) * HH
                    sendstage_ref[i, 0] = lax.slice(
                        kb[bt], (0, offt), (SKV_LOC, offt + HH))
                    sendstage_ref[i, 1] = lax.slice(
                        vb[bt], (0, offt), (SKV_LOC, offt + HH))
                kvq_ref[d, 0] = lax.slice(kb[bq], (0, off), (SKV_LOC, off + HH))
                kvq_ref[d, 1] = lax.slice(vb[bq], (0, off), (SKV_LOC, off + HH))
                for i, t in enumerate(others):
                    pltpu.make_async_remote_copy(
                        src_ref=sendstage_ref.at[i], dst_ref=kvq_ref.at[d],
                        send_sem=ssem1.at[i], recv_sem=rsem1.at[d],
                        device_id=(t,), device_id_type=pl.DeviceIdType.MESH,
                    ).start()

            wqh = lax.slice(wq, (0, off), (512, off + HH))
            xb = x_ref[bq].astype(jnp.bfloat16)
            q = (lax.dot_general(
                xb, wqh, NN, preferred_element_type=jnp.float32)
                * SCALE).astype(jnp.bfloat16)

            first, second = {0: (0, 1), 1: (1, 0), 2: (1, 0), 3: (0, 1)}[d]
            if d >= 2:
                recv_block(first)

            def blk(slot, h_local):
                kk = kvq_ref[slot, 0]
                vv = kvq_ref[slot, 1]
                sl = (0, h_local * DH)
                sh = (SKV_LOC, (h_local + 1) * DH)
                kbh = lax.slice(kk, sl, sh)
                vbh = lax.slice(vv, sl, sh)
                qh = lax.slice(q, (0, h_local * DH), (SQ, (h_local + 1) * DH))
                s = lax.dot_general(
                    qh, kbh, NT, preferred_element_type=jnp.float32)
                w = jnp.exp(s)
                if slot == 1:
                    w = jnp.where(tri, w, 0.0)
                denom = jnp.sum(w, axis=1, keepdims=True)
                ctx = lax.dot_general(
                    w.astype(jnp.bfloat16), vbh, NN,
                    preferred_element_type=jnp.float32)
                return ctx, denom

            acc = [blk(first, hl) for hl in range(2)]
            recv_block(second)
            for hl in range(2):
                c0, dn0 = acc[hl]
                c1, dn1 = blk(second, hl)
                ctx = (c0 + c1) / (dn0 + dn1)
                ctxq_ref[d, :, pl.ds(hl * DH, DH)] = ctx.astype(jnp.bfloat16)

            for j, t in enumerate(others):
                pltpu.make_async_remote_copy(
                    src_ref=ctxq_ref.at[d], dst_ref=ctxq_ref.at[d],
                    send_sem=ssem2.at[j], recv_sem=rsem2.at[d],
                    device_id=(t,), device_id_type=pl.DeviceIdType.MESH,
                ).start()

            wo_half = lax.slice(wo, (off, 0), (off + HH, SKV))
            o_mine = lax.dot_general(
                ctxq_ref[d], wo_half, NN,
                preferred_element_type=jnp.float32)

            for s in others:
                pltpu.make_async_remote_copy(
                    src_ref=ctxq_ref.at[s], dst_ref=ctxq_ref.at[s],
                    send_sem=ssem2.at[0], recv_sem=rsem2.at[s],
                    device_id=(0,), device_id_type=pl.DeviceIdType.MESH,
                ).wait_recv()

            sib = 2 * bq + (1 - half)
            off_sib = (1 - half) * HH
            wo_sib = lax.slice(wo, (off_sib, 0), (off_sib + HH, SKV))
            o_b = o_mine + lax.dot_general(
                ctxq_ref[sib], wo_sib, NN,
                preferred_element_type=jnp.float32)
            ob = 1 - bq
            cb_other = jnp.concatenate(
                [ctxq_ref[2 * ob], ctxq_ref[2 * ob + 1]], axis=1)
            o_ob = lax.dot_general(
                cb_other, wo, NN, preferred_element_type=jnp.float32)
            out_ref[bq] = o_b
            out_ref[ob] = o_ob

            if d in (0, 1):
                for j in range(3):
                    pltpu.make_async_remote_copy(
                        src_ref=sendstage_ref.at[j], dst_ref=kvq_ref.at[d],
                        send_sem=ssem1.at[j], recv_sem=rsem1.at[d],
                        device_id=(0,), device_id_type=pl.DeviceIdType.MESH,
                    ).wait_send()
            for j in range(3):
                pltpu.make_async_remote_copy(
                    src_ref=ctxq_ref.at[d], dst_ref=ctxq_ref.at[d],
                    send_sem=ssem2.at[j], recv_sem=rsem2.at[d],
                    device_id=(0,), device_id_type=pl.DeviceIdType.MESH,
                ).wait_send()

        for d in range(N_DEV):
            @pl.when(my == d)
            def _(d=d):
                run_role(d)

    return pl.pallas_call(
        body,
        out_shape=jax.ShapeDtypeStruct((B, SQ, SKV), jnp.float32),
        in_specs=[pl.BlockSpec(memory_space=pltpu.VMEM)] * 5,
        out_specs=pl.BlockSpec(memory_space=pltpu.VMEM),
        scratch_shapes=[
            pltpu.VMEM((3, 2, SKV_LOC, HH), jnp.bfloat16),
            pltpu.VMEM((2, 2, SKV_LOC, HH), jnp.bfloat16),
            pltpu.VMEM((N_DEV, SQ, HH), jnp.bfloat16),
            pltpu.SemaphoreType.DMA((3,)),
            pltpu.SemaphoreType.DMA((3,)),
            pltpu.SemaphoreType.DMA((2,)),
            pltpu.SemaphoreType.DMA((4,)),
        ],
        compiler_params=pltpu.CompilerParams(collective_id=0),
    )(x, Wq, K2, V2, Wo)
